# baseline (speedup 1.0000x reference)
import functools

import jax
import jax.numpy as jnp
from jax import lax
from jax.experimental import pallas as pl
from jax.experimental.pallas import tpu as pltpu
from jax.experimental.pallas import tpu_sc as plsc

DIM = 64
PITCH = 128
B = 16384
NC, NS, L = 2, 16, 16
NW = NC * NS
RPW = B // NW
CHUNK = 128
NCH = RPW // CHUNK
NG = CHUNK // L
EPS = 1e-6
MAGIC = 0x5F3759DF


def _sqrt16(x):
    x = jnp.maximum(x, 1e-24)
    i = plsc.bitcast(x, jnp.int32)
    y = plsc.bitcast(MAGIC - (i >> 1), jnp.float32)
    for _ in range(3):
        y = y * (1.5 - 0.5 * x * y * y)
    return x * y


_mesh = plsc.VectorSubcoreMesh(core_axis_name="c", subcore_axis_name="s",
                               num_cores=NC, num_subcores=NS)


@functools.partial(
    pl.kernel,
    out_type=jax.ShapeDtypeStruct((NW * L,), jnp.float32),
    mesh=_mesh,
    scratch_types=[
        pltpu.VMEM((NCH, CHUNK), jnp.int32),
        pltpu.VMEM((NCH, CHUNK), jnp.int32),
        pltpu.VMEM((NCH, CHUNK), jnp.int32),
        pltpu.VMEM((NCH, CHUNK), jnp.int32),
        pltpu.VMEM((NCH, CHUNK), jnp.int32),
        pltpu.VMEM((NCH, CHUNK), jnp.int32),
        pltpu.VMEM((NCH, CHUNK), jnp.int32),
        pltpu.VMEM((NCH, CHUNK), jnp.int32),
        pltpu.VMEM((NCH, CHUNK), jnp.int32),
        pltpu.VMEM((NCH, CHUNK), jnp.int32),
        pltpu.VMEM((2, CHUNK, PITCH), jnp.float32),
        pltpu.VMEM((2, CHUNK, PITCH), jnp.float32),
        pltpu.VMEM((RPW,), jnp.float32),
        pltpu.VMEM((3, PITCH), jnp.float32),
        pltpu.VMEM((3, PITCH), jnp.float32),
        pltpu.VMEM((L,), jnp.float32),
        pltpu.VMEM((L,), jnp.float32),
        pltpu.VMEM((L,), jnp.float32),
        pltpu.SemaphoreType.DMA,
        pltpu.SemaphoreType.DMA,
    ],
    compiler_params=pltpu.CompilerParams(needs_layout_passes=False,
                                         use_tc_tiling_on_sc=True),
)
def _transh_sc(ph, pt, pr, nh, nt, nr, hemb, temb, remb, rhyp, out,
               phv, ptv, prv, nhv, ntv, nrv, phi, pti, nhi, nti,
               hbuf, tbuf, poss, rhv, relv, k1r, k4r, stash, sem0, sem1):
    wid = lax.axis_index("s") * NC + lax.axis_index("c")
    sems = (sem0, sem1)
    zero = jnp.zeros((L,), jnp.float32)
    iota = lax.iota(jnp.int32, L)

    pltpu.sync_copy(ph.at[pl.ds(wid * NCH, NCH)], phv)
    pltpu.sync_copy(pt.at[pl.ds(wid * NCH, NCH)], ptv)
    pltpu.sync_copy(pr.at[pl.ds(wid * NCH, NCH)], prv)
    pltpu.sync_copy(nh.at[pl.ds(wid * NCH, NCH)], nhv)
    pltpu.sync_copy(nt.at[pl.ds(wid * NCH, NCH)], ntv)
    pltpu.sync_copy(nr.at[pl.ds(wid * NCH, NCH)], nrv)

    for raw, half in ((phv, phi), (ptv, pti), (nhv, nhi), (ntv, nti)):
        for j in range(NCH):
            for g in range(NG):
                sl = (j, pl.ds(g * L, L))
                half[sl] = raw[sl] >> 1

    phases = [(0, j) for j in range(NCH)] + [(1, j) for j in range(NCH)]

    def fire(p, slot):
        side, j = phases[p]
        hv = phi if side == 0 else nhi
        tv = pti if side == 0 else nti
        return (
            pltpu.async_copy(hemb.at[hv.at[j]], hbuf.at[slot], sems[slot]),
            pltpu.async_copy(temb.at[tv.at[j]], tbuf.at[slot], sems[slot]),
        )

    handles = {0: fire(0, 0), 1: fire(1, 1)}

    pltpu.sync_copy(rhyp, rhv)
    pltpu.sync_copy(remb, relv)
    idx6 = jnp.minimum(iota, 5)
    r6h = idx6 >> 1
    r6p = (idx6 & 1) * DIM

    def kbody(d, carry):
        k1, k4 = carry
        dv = r6p + d
        rh = plsc.load_gather(rhv, [r6h, dv])
        re = plsc.load_gather(relv, [r6h, dv])
        return k1 + re * rh, k4 + rh

    k1, k4 = lax.fori_loop(0, DIM, kbody, (zero, zero))
    k1r[...] = k1
    k4r[...] = k4

    acc_loss = zero
    for p in range(2 * NCH):
        slot = p % 2
        side, j = phases[p]
        for h in handles.pop(p):
            h.wait()
        rv, hv, tv = (prv, phv, ptv) if side == 0 else (nrv, nhv, ntv)
        slot_v = jnp.full((L,), slot, jnp.int32)
        for g in range(NG):
            sl = (j, pl.ds(g * L, L))
            r_ids = rv[sl]
            rpar = (r_ids & 1) * DIM
            rhalf = r_ids >> 1
            hpar = (hv[sl] & 1) * DIM
            tpar = (tv[sl] & 1) * DIM
            rows = g * L + iota

            def dbody(d0, carry, rows=rows, slot_v=slot_v, rhalf=rhalf,
                      rpar=rpar, hpar=hpar, tpar=tpar):
                su, su2, urh = carry
                for q in range(4):
                    dv = (d0 * 4 + q + iota) & (DIM - 1)
                    hd = plsc.load_gather(hbuf, [slot_v, rows, dv + hpar])
                    td = plsc.load_gather(tbuf, [slot_v, rows, dv + tpar])
                    rh = plsc.load_gather(rhv, [rhalf, dv + rpar])
                    re = plsc.load_gather(relv, [rhalf, dv + rpar])
                    u = hd - td + re
                    su = su + u
                    su2 = su2 + u * u
                    urh = urh + u * rh
                return su, su2, urh

            su, su2, urh = lax.fori_loop(0, DIM // 4, dbody,
                                         (zero, zero, zero))
            c = urh - plsc.load_gather(k1r, [r_ids])
            k4g = plsc.load_gather(k4r, [r_ids])
            s = (su2 - 2.0 * c * urh + c * c
                 + (2.0 * EPS) * (su - c * k4g) + DIM * EPS * EPS)
            score = _sqrt16(s)
            off = j * CHUNK + g * L
            if side == 0:
                poss[pl.ds(off, L)] = score
            else:
                psc = poss[pl.ds(off, L)]
                acc_loss = acc_loss + jnp.maximum(psc - score + 1.0, 0.0)
        if p + 2 < 2 * NCH:
            handles[p + 2] = fire(p + 2, slot)

    stash[...] = jnp.full((L,), jnp.sum(acc_loss))
    pltpu.sync_copy(stash, out.at[pl.ds(wid * L, L)])


def kernel(posX, negX, H_emb, T_emb, R_emb, R_hyper):
    nrows = B // CHUNK
    ph = posX[:, 0].reshape(nrows, CHUNK)
    pt = posX[:, 1].reshape(nrows, CHUNK)
    pr = posX[:, 2].reshape(nrows, CHUNK)
    nh = negX[:, 0].reshape(nrows, CHUNK)
    nt = negX[:, 1].reshape(nrows, CHUNK)
    nr = negX[:, 2].reshape(nrows, CHUNK)
    def pairs(tbl):
        return jnp.concatenate([tbl[0::2], tbl[1::2]], axis=1)

    partials = _transh_sc(ph, pt, pr, nh, nt, nr,
                          pairs(H_emb), pairs(T_emb),
                          R_emb.reshape(3, PITCH),
                          R_hyper.reshape(3, PITCH))
    return jnp.sum(partials.reshape(NW, L)[:, 0]) / posX.shape[0]


# speedup vs baseline: 11.4393x; 11.4393x over previous
import functools

import jax
import jax.numpy as jnp
from jax import lax
from jax.experimental import pallas as pl
from jax.experimental.pallas import tpu as pltpu
from jax.experimental.pallas import tpu_sc as plsc

DIM = 64
B = 16384
NC, NS, L = 2, 16, 16
NW = NC * NS
RPW = B // NW
CHUNK = 128
NCH = RPW // CHUNK
NG = CHUNK // L
UNROLL = 4
EPS = 1e-6
MAGIC = 0x5F3759DF


def _sqrt16(x):
    x = jnp.maximum(x, 1e-24)
    i = plsc.bitcast(x, jnp.int32)
    y = plsc.bitcast(MAGIC - (i >> 1), jnp.float32)
    for _ in range(3):
        y = y * (1.5 - 0.5 * x * y * y)
    return x * y


_mesh = plsc.VectorSubcoreMesh(core_axis_name="c", subcore_axis_name="s",
                               num_cores=NC, num_subcores=NS)


@functools.partial(
    pl.kernel,
    out_type=jax.ShapeDtypeStruct((NW * L,), jnp.float32),
    mesh=_mesh,
    scratch_types=[
        pltpu.VMEM((NCH, CHUNK), jnp.int32),
        pltpu.VMEM((NCH, CHUNK), jnp.int32),
        pltpu.VMEM((NCH, CHUNK), jnp.int32),
        pltpu.VMEM((NCH, CHUNK), jnp.int32),
        pltpu.VMEM((NCH, CHUNK), jnp.int32),
        pltpu.VMEM((NCH, CHUNK), jnp.int32),
        pltpu.VMEM((2, CHUNK, DIM), jnp.float32),
        pltpu.VMEM((2, CHUNK, DIM), jnp.float32),
        pltpu.VMEM((RPW,), jnp.float32),
        pltpu.VMEM((6, DIM), jnp.float32),
        pltpu.VMEM((6, DIM), jnp.float32),
        pltpu.VMEM((L,), jnp.float32),
        pltpu.VMEM((L,), jnp.float32),
        pltpu.VMEM((L,), jnp.float32),
        pltpu.SemaphoreType.DMA,
        pltpu.SemaphoreType.DMA,
    ],
    compiler_params=pltpu.CompilerParams(needs_layout_passes=False,
                                         use_tc_tiling_on_sc=False),
)
def _transh_sc(ph, pt, pr, nh, nt, nr, hemb, temb, remb, rhyp, out,
               phv, ptv, prv, nhv, ntv, nrv, hbuf, tbuf, poss,
               rhv, relv, k1r, k4r, stash, sem0, sem1):
    wid = lax.axis_index("s") * NC + lax.axis_index("c")
    sems = (sem0, sem1)
    zero = jnp.zeros((L,), jnp.float32)
    iota = lax.iota(jnp.int32, L)

    pltpu.sync_copy(ph.at[pl.ds(wid * NCH, NCH)], phv)
    pltpu.sync_copy(pt.at[pl.ds(wid * NCH, NCH)], ptv)
    pltpu.sync_copy(pr.at[pl.ds(wid * NCH, NCH)], prv)
    pltpu.sync_copy(nh.at[pl.ds(wid * NCH, NCH)], nhv)
    pltpu.sync_copy(nt.at[pl.ds(wid * NCH, NCH)], ntv)
    pltpu.sync_copy(nr.at[pl.ds(wid * NCH, NCH)], nrv)

    phases = [(0, j) for j in range(NCH)] + [(1, j) for j in range(NCH)]

    def fire(p, slot):
        side, j = phases[p]
        hv = phv if side == 0 else nhv
        tv = ptv if side == 0 else ntv
        return (
            pltpu.async_copy(hemb.at[hv.at[j]], hbuf.at[slot], sems[slot]),
            pltpu.async_copy(temb.at[tv.at[j]], tbuf.at[slot], sems[slot]),
        )

    handles = {0: fire(0, 0), 1: fire(1, 1)}

    pltpu.sync_copy(rhyp, rhv)
    pltpu.sync_copy(remb, relv)
    idx6 = jnp.minimum(iota, 5)

    def kbody(d, carry):
        k1, k4 = carry
        dv = jnp.full((L,), d, jnp.int32)
        rh = plsc.load_gather(rhv, [idx6, dv])
        re = plsc.load_gather(relv, [idx6, dv])
        return k1 + re * rh, k4 + rh

    k1, k4 = lax.fori_loop(0, DIM, kbody, (zero, zero))
    k1r[...] = k1
    k4r[...] = k4

    acc_loss = zero
    for p in range(2 * NCH):
        slot = p % 2
        side, j = phases[p]
        for h in handles.pop(p):
            h.wait()
        rv = prv if side == 0 else nrv
        slot_v = jnp.full((L,), slot, jnp.int32)
        for g in range(NG):
            r_ids = rv[j, pl.ds(g * L, L)]
            rows = g * L + iota

            def dbody(d0, carry, rows=rows, slot_v=slot_v, r_ids=r_ids):
                su, su2, urh = carry
                for q in range(UNROLL):
                    dv = (d0 * UNROLL + q + iota) & (DIM - 1)
                    hd = plsc.load_gather(hbuf, [slot_v, rows, dv])
                    td = plsc.load_gather(tbuf, [slot_v, rows, dv])
                    rh = plsc.load_gather(rhv, [r_ids, dv])
                    re = plsc.load_gather(relv, [r_ids, dv])
                    u = hd - td + re
                    su = su + u
                    su2 = su2 + u * u
                    urh = urh + u * rh
                return su, su2, urh

            su, su2, urh = lax.fori_loop(0, DIM // UNROLL, dbody,
                                         (zero, zero, zero))
            c = urh - plsc.load_gather(k1r, [r_ids])
            k4g = plsc.load_gather(k4r, [r_ids])
            s = (su2 - 2.0 * c * urh + c * c
                 + (2.0 * EPS) * (su - c * k4g) + DIM * EPS * EPS)
            score = _sqrt16(s)
            off = j * CHUNK + g * L
            if side == 0:
                poss[pl.ds(off, L)] = score
            else:
                psc = poss[pl.ds(off, L)]
                acc_loss = acc_loss + jnp.maximum(psc - score + 1.0, 0.0)
        if p + 2 < 2 * NCH:
            handles[p + 2] = fire(p + 2, slot)

    stash[...] = jnp.full((L,), jnp.sum(acc_loss))
    pltpu.sync_copy(stash, out.at[pl.ds(wid * L, L)])


def kernel(posX, negX, H_emb, T_emb, R_emb, R_hyper):
    nrows = B // CHUNK
    ph = posX[:, 0].reshape(nrows, CHUNK)
    pt = posX[:, 1].reshape(nrows, CHUNK)
    pr = posX[:, 2].reshape(nrows, CHUNK)
    nh = negX[:, 0].reshape(nrows, CHUNK)
    nt = negX[:, 1].reshape(nrows, CHUNK)
    nr = negX[:, 2].reshape(nrows, CHUNK)
    partials = _transh_sc(ph, pt, pr, nh, nt, nr,
                          H_emb, T_emb, R_emb, R_hyper)
    return jnp.sum(partials.reshape(NW, L)[:, 0]) / posX.shape[0]


# speedup vs baseline: 11.7532x; 1.0274x over previous
import functools

import jax
import jax.numpy as jnp
from jax import lax
from jax.experimental import pallas as pl
from jax.experimental.pallas import tpu as pltpu
from jax.experimental.pallas import tpu_sc as plsc

DIM = 64
B = 16384
NC, NS, L = 2, 16, 16
NW = NC * NS
RPW = B // NW
CHUNK = 128
NCH = RPW // CHUNK
NG = CHUNK // L
UNROLL = 8
EPS = 1e-6
MAGIC = 0x5F3759DF


def _sqrt16(x):
    x = jnp.maximum(x, 1e-24)
    i = plsc.bitcast(x, jnp.int32)
    y = plsc.bitcast(MAGIC - (i >> 1), jnp.float32)
    for _ in range(3):
        y = y * (1.5 - 0.5 * x * y * y)
    return x * y


_mesh = plsc.VectorSubcoreMesh(core_axis_name="c", subcore_axis_name="s",
                               num_cores=NC, num_subcores=NS)


@functools.partial(
    pl.kernel,
    out_type=jax.ShapeDtypeStruct((NW * L,), jnp.float32),
    mesh=_mesh,
    scratch_types=[
        pltpu.VMEM((NCH, CHUNK), jnp.int32),
        pltpu.VMEM((NCH, CHUNK), jnp.int32),
        pltpu.VMEM((NCH, CHUNK), jnp.int32),
        pltpu.VMEM((NCH, CHUNK), jnp.int32),
        pltpu.VMEM((NCH, CHUNK), jnp.int32),
        pltpu.VMEM((NCH, CHUNK), jnp.int32),
        pltpu.VMEM((2, CHUNK, DIM), jnp.float32),
        pltpu.VMEM((2, CHUNK, DIM), jnp.float32),
        pltpu.VMEM((RPW,), jnp.float32),
        pltpu.VMEM((6, DIM), jnp.float32),
        pltpu.VMEM((6, DIM), jnp.float32),
        pltpu.VMEM((L,), jnp.float32),
        pltpu.VMEM((L,), jnp.float32),
        pltpu.VMEM((L,), jnp.float32),
        pltpu.SemaphoreType.DMA,
        pltpu.SemaphoreType.DMA,
    ],
    compiler_params=pltpu.CompilerParams(needs_layout_passes=False,
                                         use_tc_tiling_on_sc=False),
)
def _transh_sc(ph, pt, pr, nh, nt, nr, hemb, temb, remb, rhyp, out,
               phv, ptv, prv, nhv, ntv, nrv, hbuf, tbuf, poss,
               rhv, relv, k1r, k4r, stash, sem0, sem1):
    wid = lax.axis_index("s") * NC + lax.axis_index("c")
    sems = (sem0, sem1)
    zero = jnp.zeros((L,), jnp.float32)
    iota = lax.iota(jnp.int32, L)

    pltpu.sync_copy(ph.at[pl.ds(wid * NCH, NCH)], phv)
    pltpu.sync_copy(pt.at[pl.ds(wid * NCH, NCH)], ptv)
    pltpu.sync_copy(pr.at[pl.ds(wid * NCH, NCH)], prv)
    pltpu.sync_copy(nh.at[pl.ds(wid * NCH, NCH)], nhv)
    pltpu.sync_copy(nt.at[pl.ds(wid * NCH, NCH)], ntv)
    pltpu.sync_copy(nr.at[pl.ds(wid * NCH, NCH)], nrv)

    phases = [(0, j) for j in range(NCH)] + [(1, j) for j in range(NCH)]

    def fire(p, slot):
        side, j = phases[p]
        hv = phv if side == 0 else nhv
        tv = ptv if side == 0 else ntv
        return (
            pltpu.async_copy(hemb.at[hv.at[j]], hbuf.at[slot], sems[slot]),
            pltpu.async_copy(temb.at[tv.at[j]], tbuf.at[slot], sems[slot]),
        )

    handles = {0: fire(0, 0), 1: fire(1, 1)}

    pltpu.sync_copy(rhyp, rhv)
    pltpu.sync_copy(remb, relv)
    idx6 = jnp.minimum(iota, 5)

    def kbody(d, carry):
        k1, k4 = carry
        dv = jnp.full((L,), d, jnp.int32)
        rh = plsc.load_gather(rhv, [idx6, dv])
        re = plsc.load_gather(relv, [idx6, dv])
        return k1 + re * rh, k4 + rh

    k1, k4 = lax.fori_loop(0, DIM, kbody, (zero, zero))
    k1r[...] = k1
    k4r[...] = k4

    acc_loss = zero
    for p in range(2 * NCH):
        slot = p % 2
        side, j = phases[p]
        for h in handles.pop(p):
            h.wait()
        rv = prv if side == 0 else nrv
        slot_v = jnp.full((L,), slot, jnp.int32)

        def gbody(g, acc, side=side, j=j, slot_v=slot_v, rv=rv):
            r_ids = rv[j, pl.ds(g * L, L)]
            rows = g * L + iota
            def dbody(d0, carry, rows=rows, r_ids=r_ids, slot_v=slot_v):
                su, su2, urh = carry
                for q in range(UNROLL):
                    dv = (d0 * UNROLL + q + iota) & (DIM - 1)
                    hd = plsc.load_gather(hbuf, [slot_v, rows, dv])
                    td = plsc.load_gather(tbuf, [slot_v, rows, dv])
                    rh = plsc.load_gather(rhv, [r_ids, dv])
                    re = plsc.load_gather(relv, [r_ids, dv])
                    u = hd - td + re
                    su = su + u
                    su2 = su2 + u * u
                    urh = urh + u * rh
                return su, su2, urh

            su, su2, urh = lax.fori_loop(0, DIM // UNROLL, dbody,
                                         (zero, zero, zero))
            c = urh - plsc.load_gather(k1r, [r_ids])
            k4g = plsc.load_gather(k4r, [r_ids])
            s = (su2 - 2.0 * c * urh + c * c
                 + (2.0 * EPS) * (su - c * k4g) + DIM * EPS * EPS)
            score = _sqrt16(s)
            off = j * CHUNK + g * L
            if side == 0:
                poss[pl.ds(off, L)] = score
                return acc
            psc = poss[pl.ds(off, L)]
            return acc + jnp.maximum(psc - score + 1.0, 0.0)

        acc_loss = lax.fori_loop(0, NG, gbody, acc_loss)
        if p + 2 < 2 * NCH:
            handles[p + 2] = fire(p + 2, slot)

    stash[...] = jnp.full((L,), jnp.sum(acc_loss))
    pltpu.sync_copy(stash, out.at[pl.ds(wid * L, L)])


def kernel(posX, negX, H_emb, T_emb, R_emb, R_hyper):
    nrows = B // CHUNK
    ph = posX[:, 0].reshape(nrows, CHUNK)
    pt = posX[:, 1].reshape(nrows, CHUNK)
    pr = posX[:, 2].reshape(nrows, CHUNK)
    nh = negX[:, 0].reshape(nrows, CHUNK)
    nt = negX[:, 1].reshape(nrows, CHUNK)
    nr = negX[:, 2].reshape(nrows, CHUNK)
    partials = _transh_sc(ph, pt, pr, nh, nt, nr,
                          H_emb, T_emb, R_emb, R_hyper)
    return jnp.sum(partials.reshape(NW, L)[:, 0]) / posX.shape[0]


# speedup vs baseline: 11.8131x; 1.0051x over previous
import functools

import jax
import jax.numpy as jnp
from jax import lax
from jax.experimental import pallas as pl
from jax.experimental.pallas import tpu as pltpu
from jax.experimental.pallas import tpu_sc as plsc

DIM = 64
B = 16384
NC, NS, L = 2, 16, 16
NW = NC * NS
RPW = B // NW
CHUNK = 128
NCH = RPW // CHUNK
NG = CHUNK // L
UNROLL = 8
EPS = 1e-6
MAGIC = 0x5F3759DF


def _sqrt16(x):
    x = jnp.maximum(x, 1e-24)
    i = plsc.bitcast(x, jnp.int32)
    y = plsc.bitcast(MAGIC - (i >> 1), jnp.float32)
    for _ in range(3):
        y = y * (1.5 - 0.5 * x * y * y)
    return x * y


_mesh = plsc.VectorSubcoreMesh(core_axis_name="c", subcore_axis_name="s",
                               num_cores=NC, num_subcores=NS)


@functools.partial(
    pl.kernel,
    out_type=jax.ShapeDtypeStruct((NW * L,), jnp.float32),
    mesh=_mesh,
    scratch_types=[
        pltpu.VMEM((NCH, CHUNK), jnp.int32),
        pltpu.VMEM((NCH, CHUNK), jnp.int32),
        pltpu.VMEM((NCH, CHUNK), jnp.int32),
        pltpu.VMEM((NCH, CHUNK), jnp.int32),
        pltpu.VMEM((NCH, CHUNK), jnp.int32),
        pltpu.VMEM((NCH, CHUNK), jnp.int32),
        pltpu.VMEM((2, CHUNK, DIM), jnp.float32),
        pltpu.VMEM((2, CHUNK, DIM), jnp.float32),
        pltpu.VMEM((RPW,), jnp.float32),
        pltpu.VMEM((6, DIM), jnp.float32),
        pltpu.VMEM((6, DIM), jnp.float32),
        pltpu.VMEM((L,), jnp.float32),
        pltpu.VMEM((L,), jnp.float32),
        pltpu.VMEM((L,), jnp.float32),
        pltpu.SemaphoreType.DMA,
        pltpu.SemaphoreType.DMA,
    ],
    compiler_params=pltpu.CompilerParams(needs_layout_passes=False,
                                         use_tc_tiling_on_sc=False),
)
def _transh_sc(ph, pt, pr, nh, nt, nr, hemb, temb, remb, rhyp, out,
               phv, ptv, prv, nhv, ntv, nrv, hbuf, tbuf, poss,
               rhv, relv, k1r, k4r, stash, sem0, sem1):
    wid = lax.axis_index("s") * NC + lax.axis_index("c")
    sems = (sem0, sem1)
    zero = jnp.zeros((L,), jnp.float32)
    iota = lax.iota(jnp.int32, L)

    pltpu.sync_copy(ph.at[pl.ds(wid * NCH, NCH)], phv)
    pltpu.sync_copy(pt.at[pl.ds(wid * NCH, NCH)], ptv)

    phases = [(0, j) for j in range(NCH)] + [(1, j) for j in range(NCH)]

    def fire(p, slot):
        side, j = phases[p]
        hv = phv if side == 0 else nhv
        tv = ptv if side == 0 else ntv
        return (
            pltpu.async_copy(hemb.at[hv.at[j]], hbuf.at[slot], sems[slot]),
            pltpu.async_copy(temb.at[tv.at[j]], tbuf.at[slot], sems[slot]),
        )

    handles = {0: fire(0, 0), 1: fire(1, 1)}

    pltpu.sync_copy(pr.at[pl.ds(wid * NCH, NCH)], prv)
    pltpu.sync_copy(nh.at[pl.ds(wid * NCH, NCH)], nhv)
    pltpu.sync_copy(nt.at[pl.ds(wid * NCH, NCH)], ntv)
    pltpu.sync_copy(nr.at[pl.ds(wid * NCH, NCH)], nrv)

    pltpu.sync_copy(rhyp, rhv)
    pltpu.sync_copy(remb, relv)
    idx6 = jnp.minimum(iota, 5)

    def kbody(d, carry):
        k1, k4 = carry
        dv = jnp.full((L,), d, jnp.int32)
        rh = plsc.load_gather(rhv, [idx6, dv])
        re = plsc.load_gather(relv, [idx6, dv])
        return k1 + re * rh, k4 + rh

    k1, k4 = lax.fori_loop(0, DIM, kbody, (zero, zero))
    k1r[...] = k1
    k4r[...] = k4

    acc_loss = zero
    for p in range(2 * NCH):
        slot = p % 2
        side, j = phases[p]
        for h in handles.pop(p):
            h.wait()
        rv = prv if side == 0 else nrv
        slot_v = jnp.full((L,), slot, jnp.int32)

        def gbody(g, acc, side=side, j=j, slot_v=slot_v, rv=rv):
            r_ids = rv[j, pl.ds(g * L, L)]
            rows = g * L + iota
            def dbody(d0, carry, rows=rows, r_ids=r_ids, slot_v=slot_v):
                su, su2, urh = carry
                for q in range(UNROLL):
                    dv = (d0 * UNROLL + q + iota) & (DIM - 1)
                    hd = plsc.load_gather(hbuf, [slot_v, rows, dv])
                    td = plsc.load_gather(tbuf, [slot_v, rows, dv])
                    rh = plsc.load_gather(rhv, [r_ids, dv])
                    re = plsc.load_gather(relv, [r_ids, dv])
                    u = hd - td + re
                    su = su + u
                    su2 = su2 + u * u
                    urh = urh + u * rh
                return su, su2, urh

            su, su2, urh = lax.fori_loop(0, DIM // UNROLL, dbody,
                                         (zero, zero, zero))
            c = urh - plsc.load_gather(k1r, [r_ids])
            k4g = plsc.load_gather(k4r, [r_ids])
            s = (su2 - 2.0 * c * urh + c * c
                 + (2.0 * EPS) * (su - c * k4g) + DIM * EPS * EPS)
            score = _sqrt16(s)
            off = j * CHUNK + g * L
            if side == 0:
                poss[pl.ds(off, L)] = score
                return acc
            psc = poss[pl.ds(off, L)]
            return acc + jnp.maximum(psc - score + 1.0, 0.0)

        acc_loss = lax.fori_loop(0, NG, gbody, acc_loss)
        if p + 2 < 2 * NCH:
            handles[p + 2] = fire(p + 2, slot)

    stash[...] = jnp.full((L,), jnp.sum(acc_loss))
    pltpu.sync_copy(stash, out.at[pl.ds(wid * L, L)])


def kernel(posX, negX, H_emb, T_emb, R_emb, R_hyper):
    nrows = B // CHUNK
    ph = posX[:, 0].reshape(nrows, CHUNK)
    pt = posX[:, 1].reshape(nrows, CHUNK)
    pr = posX[:, 2].reshape(nrows, CHUNK)
    nh = negX[:, 0].reshape(nrows, CHUNK)
    nt = negX[:, 1].reshape(nrows, CHUNK)
    nr = negX[:, 2].reshape(nrows, CHUNK)
    partials = _transh_sc(ph, pt, pr, nh, nt, nr,
                          H_emb, T_emb, R_emb, R_hyper)
    return jnp.sum(partials.reshape(NW, L)[:, 0]) / posX.shape[0]
